# baseline (speedup 1.0000x reference)
import functools

import jax
import jax.numpy as jnp
from jax import lax
from jax.experimental import pallas as pl
from jax.experimental.pallas import tpu as pltpu
from jax.experimental.pallas import tpu_sc as plsc

SEQ = 8192
BATCH = 4
D = 768

NUM_CORES = 2
NUM_SUBCORES = 16
NW = NUM_CORES * NUM_SUBCORES
ROWS_PER_W = SEQ // NW
NBUF = 4
CHUNK = 32
NCHUNK = ROWS_PER_W // CHUNK


def _body(pe_hbm, out_hbm, *scratch):
    bufs = scratch[:NBUF]
    rsems = scratch[NBUF:2 * NBUF]
    wsems = scratch[2 * NBUF:3 * NBUF]
    wid = lax.axis_index("s") * NUM_CORES + lax.axis_index("c")
    base = wid * ROWS_PER_W

    reads = [None] * NCHUNK
    writes = [None] * NCHUNK

    def start_read(k):
        j = k % NBUF
        reads[k] = pltpu.async_copy(
            pe_hbm.at[pl.ds(base + k * CHUNK, CHUNK)], bufs[j], rsems[j])

    start_read(0)
    for i in range(NCHUNK):
        j = i % NBUF
        if i + 1 < NCHUNK:
            if i + 1 - NBUF >= 0:
                for c in writes[i + 1 - NBUF]:
                    c.wait()
            start_read(i + 1)
        reads[i].wait()
        row0 = base + i * CHUNK
        writes[i] = [
            pltpu.async_copy(bufs[j], out_hbm.at[pl.ds(row0, CHUNK), b], wsems[j])
            for b in range(BATCH)
        ]
    for k in range(max(0, NCHUNK - NBUF), NCHUNK):
        for c in writes[k]:
            c.wait()


_bcast = functools.partial(
    pl.kernel,
    out_type=jax.ShapeDtypeStruct((SEQ, BATCH, D), jnp.float32),
    mesh=plsc.VectorSubcoreMesh(
        core_axis_name="c", subcore_axis_name="s",
        num_cores=NUM_CORES, num_subcores=NUM_SUBCORES),
    scratch_types=(
        [pltpu.VMEM((CHUNK, D), jnp.float32) for _ in range(NBUF)]
        + [pltpu.SemaphoreType.DMA for _ in range(2 * NBUF)]
    ),
)(_body)


@jax.jit
def kernel(z, pos_embed):
    del z
    return _bcast(pos_embed)


# speedup vs baseline: 1.0830x; 1.0830x over previous
import functools

import jax
import jax.numpy as jnp
from jax import lax
from jax.experimental import pallas as pl
from jax.experimental.pallas import tpu as pltpu
from jax.experimental.pallas import tpu_sc as plsc

SEQ = 8192
BATCH = 4
D = 768

NUM_CORES = 2
NUM_SUBCORES = 16
NW = NUM_CORES * NUM_SUBCORES
ROWS_PER_W = SEQ // NW
NBUF = 2
CHUNK = 64
NCHUNK = ROWS_PER_W // CHUNK


def _body(pe_hbm, out_hbm, *scratch):
    bufs = scratch[:NBUF]
    rsems = scratch[NBUF:2 * NBUF]
    wsems = scratch[2 * NBUF:3 * NBUF]
    wid = lax.axis_index("s") * NUM_CORES + lax.axis_index("c")
    base = wid * ROWS_PER_W

    reads = [None] * NCHUNK
    writes = [None] * NCHUNK

    def start_read(k):
        j = k % NBUF
        reads[k] = pltpu.async_copy(
            pe_hbm.at[pl.ds(base + k * CHUNK, CHUNK)], bufs[j], rsems[j])

    start_read(0)
    for i in range(NCHUNK):
        j = i % NBUF
        if i + 1 < NCHUNK:
            if i + 1 - NBUF >= 0:
                for c in writes[i + 1 - NBUF]:
                    c.wait()
            start_read(i + 1)
        reads[i].wait()
        row0 = base + i * CHUNK
        writes[i] = [
            pltpu.async_copy(bufs[j], out_hbm.at[pl.ds(row0, CHUNK), b], wsems[j])
            for b in range(BATCH)
        ]
    for k in range(max(0, NCHUNK - NBUF), NCHUNK):
        for c in writes[k]:
            c.wait()


_bcast = functools.partial(
    pl.kernel,
    out_type=jax.ShapeDtypeStruct((SEQ, BATCH, D), jnp.float32),
    mesh=plsc.VectorSubcoreMesh(
        core_axis_name="c", subcore_axis_name="s",
        num_cores=NUM_CORES, num_subcores=NUM_SUBCORES),
    scratch_types=(
        [pltpu.VMEM((CHUNK, D), jnp.float32) for _ in range(NBUF)]
        + [pltpu.SemaphoreType.DMA for _ in range(2 * NBUF)]
    ),
)(_body)


@jax.jit
def kernel(z, pos_embed):
    del z
    return _bcast(pos_embed)
